# Initial kernel scaffold; baseline (speedup 1.0000x reference)
import jax
import jax.numpy as jnp
from jax.experimental import pallas as pl


def kernel(polylines, polylines_mask, W0, g0, b0, W1, g1, b1, W2, g2, b2, W3, b3, W4, b4):
    raise NotImplementedError("write your pallas kernel here")


# speedup vs baseline: 1.8893x; 1.8893x over previous
import jax
import jax.numpy as jnp
from jax.experimental import pallas as pl
from jax.experimental.pallas import tpu as pltpu

EPS = 1e-5


def _encoder_kernel(x_ref, m_ref, w0_ref, b0_ref, w1a_ref, w1b_ref, b1_ref,
                    w2_ref, b2_ref, w3_ref, b3_ref, w4_ref, b4_ref,
                    out_ref, f_scratch):
    P = x_ref.shape[0]
    w0 = w0_ref[...]
    b0 = b0_ref[...]
    pooled = None
    vsum = None
    for p in range(P):
        xp = x_ref[p]
        mp = m_ref[p]
        f = jnp.dot(xp, w0, preferred_element_type=jnp.float32) + b0
        f = jnp.maximum(f, 0.0) * mp
        f_scratch[p] = f
        pooled = f if pooled is None else jnp.maximum(pooled, f)
        vsum = mp if vsum is None else vsum + mp
    pw = jnp.dot(pooled, w1b_ref[...], preferred_element_type=jnp.float32)
    pw = pw + b1_ref[...]
    w1a = w1a_ref[...]
    w2 = w2_ref[...]
    b2 = b2_ref[...]
    buf = None
    for p in range(P):
        mp = m_ref[p]
        h = jnp.dot(f_scratch[p], w1a, preferred_element_type=jnp.float32)
        h = jnp.maximum(h + pw, 0.0) * mp
        h2 = jnp.dot(h, w2, preferred_element_type=jnp.float32) + b2
        h2 = jnp.maximum(h2, 0.0) * mp
        buf = h2 if buf is None else jnp.maximum(buf, h2)
    o = jnp.dot(buf, w3_ref[...], preferred_element_type=jnp.float32)
    o = jnp.maximum(o + b3_ref[...], 0.0)
    o = jnp.dot(o, w4_ref[...], preferred_element_type=jnp.float32)
    o = o + b4_ref[...]
    out_ref[...] = o * (vsum > 0.0).astype(o.dtype)


def kernel(polylines, polylines_mask, W0, g0, b0, W1, g1, b1, W2, g2, b2,
           W3, b3, W4, b4):
    N, P, C = polylines.shape
    H = W0.shape[1]
    O = W4.shape[1]
    s = 1.0 / jnp.sqrt(jnp.float32(1.0) + EPS)
    W0s = W0 * (g0 * s)[None, :]
    W1s = W1 * (g1 * s)[None, :]
    W1a, W1b = W1s[:H], W1s[H:]
    W2s = W2 * (g2 * s)[None, :]

    xT = polylines.transpose(1, 0, 2)
    mT = polylines_mask.astype(jnp.float32).T[:, :, None]

    blk = 256
    grid = (N // blk,)
    full = lambda shape: pl.BlockSpec(shape, lambda i: (0,) * len(shape))

    return pl.pallas_call(
        _encoder_kernel,
        grid=grid,
        in_specs=[
            pl.BlockSpec((P, blk, C), lambda i: (0, i, 0)),
            pl.BlockSpec((P, blk, 1), lambda i: (0, i, 0)),
            full((C, H)),
            full((1, H)),
            full((H, H)),
            full((H, H)),
            full((1, H)),
            full((H, H)),
            full((1, H)),
            full((H, H)),
            full((1, H)),
            full((H, O)),
            full((1, O)),
        ],
        out_specs=pl.BlockSpec((blk, O), lambda i: (i, 0)),
        out_shape=jax.ShapeDtypeStruct((N, O), jnp.float32),
        scratch_shapes=[pltpu.VMEM((P, blk, H), jnp.float32)],
        compiler_params=pltpu.CompilerParams(
            dimension_semantics=("parallel",),
        ),
    )(xT, mT, W0s, b0.reshape(1, H), W1a, W1b, b1.reshape(1, H),
      W2s, b2.reshape(1, H), W3, b3.reshape(1, H), W4, b4.reshape(1, O))


# speedup vs baseline: 2.0680x; 1.0946x over previous
import jax
import jax.numpy as jnp
from jax.experimental import pallas as pl
from jax.experimental.pallas import tpu as pltpu

EPS = 1e-5


def _encoder_kernel(x_ref, m_ref, w0_ref, b0_ref, w1a_ref, w1b_ref, b1_ref,
                    w2_ref, b2_ref, w3_ref, b3_ref, w4_ref, b4_ref,
                    out_ref, f_scr):
    P, blk, C = x_ref.shape
    H = w0_ref.shape[1]
    m2d = m_ref[...]
    x2 = x_ref[...].reshape(P * blk, C)
    f2 = jnp.dot(x2, w0_ref[...], preferred_element_type=jnp.float32)
    f_scr[...] = jnp.maximum(f2 + b0_ref[...], 0.0)
    pooled = None
    for p in range(P):
        mp = m2d[:, p:p + 1]
        fp = f_scr[pl.ds(p * blk, blk), :] * mp
        pooled = fp if pooled is None else jnp.maximum(pooled, fp)
    pw = jnp.dot(pooled, w1b_ref[...], preferred_element_type=jnp.float32)
    pw = pw + b1_ref[...]
    w1a = w1a_ref[...]
    w2 = w2_ref[...]
    b2 = b2_ref[...]
    buf = None
    for p in range(P):
        mp = m2d[:, p:p + 1]
        fp = f_scr[pl.ds(p * blk, blk), :] * mp
        h = jnp.dot(fp, w1a, preferred_element_type=jnp.float32)
        h = jnp.maximum(h + pw, 0.0)
        h2 = jnp.dot(h, w2, preferred_element_type=jnp.float32) + b2
        h2 = jnp.maximum(h2, 0.0) * mp
        buf = h2 if buf is None else jnp.maximum(buf, h2)
    o = jnp.dot(buf, w3_ref[...], preferred_element_type=jnp.float32)
    o = jnp.maximum(o + b3_ref[...], 0.0)
    o = jnp.dot(o, w4_ref[...], preferred_element_type=jnp.float32)
    o = o + b4_ref[...]
    valid = jnp.max(m2d, axis=1, keepdims=True)
    out_ref[...] = o * valid


def kernel(polylines, polylines_mask, W0, g0, b0, W1, g1, b1, W2, g2, b2,
           W3, b3, W4, b4):
    N, P, C = polylines.shape
    H = W0.shape[1]
    O = W4.shape[1]
    s = 1.0 / jnp.sqrt(jnp.float32(1.0) + EPS)
    W0s = W0 * (g0 * s)[None, :]
    W1s = W1 * (g1 * s)[None, :]
    W1a, W1b = W1s[:H], W1s[H:]
    W2s = W2 * (g2 * s)[None, :]

    xT = polylines.transpose(1, 0, 2)
    mf = polylines_mask.astype(jnp.float32)

    blk = 512
    grid = (N // blk,)
    full = lambda shape: pl.BlockSpec(shape, lambda i: (0,) * len(shape))

    return pl.pallas_call(
        _encoder_kernel,
        grid=grid,
        in_specs=[
            pl.BlockSpec((P, blk, C), lambda i: (0, i, 0)),
            pl.BlockSpec((blk, P), lambda i: (i, 0)),
            full((C, H)),
            full((1, H)),
            full((H, H)),
            full((H, H)),
            full((1, H)),
            full((H, H)),
            full((1, H)),
            full((H, H)),
            full((1, H)),
            full((H, O)),
            full((1, O)),
        ],
        out_specs=pl.BlockSpec((blk, O), lambda i: (i, 0)),
        out_shape=jax.ShapeDtypeStruct((N, O), jnp.float32),
        scratch_shapes=[pltpu.VMEM((P * blk, H), jnp.float32)],
        compiler_params=pltpu.CompilerParams(
            dimension_semantics=("parallel",),
        ),
    )(xT, mf, W0s, b0.reshape(1, H), W1a, W1b, b1.reshape(1, H),
      W2s, b2.reshape(1, H), W3, b3.reshape(1, H), W4, b4.reshape(1, O))
